# baseline (speedup 1.0000x reference)
import functools

import jax
import jax.numpy as jnp
from jax import lax
from jax.experimental import pallas as pl
from jax.experimental.pallas import tpu as pltpu
from jax.experimental.pallas import tpu_sc as plsc

NUM_CORES = 2
NUM_SUBCORES = 16
NUM_WORKERS = NUM_CORES * NUM_SUBCORES
LANES = 16

CHUNK = 128
NBUF = 4


def _sc_partial_sums(ids_flat, table, B, L, D):
  ids_per_worker = (B * L) // NUM_WORKERS
  num_chunks = ids_per_worker // CHUNK
  vecs_per_row = D // LANES
  log2_b = B.bit_length() - 1
  assert (1 << log2_b) == B

  mesh = plsc.VectorSubcoreMesh(
      core_axis_name="c", subcore_axis_name="s",
      num_cores=NUM_CORES, num_subcores=NUM_SUBCORES)

  scratch = (
      [pltpu.VMEM((ids_per_worker,), jnp.int32)]
      + [pltpu.VMEM((CHUNK, D), jnp.float32) for _ in range(NBUF)]
      + [pltpu.VMEM((L, D), jnp.float32)]
      + [pltpu.SemaphoreType.DMA for _ in range(NBUF)]
  )

  @functools.partial(
      pl.kernel,
      out_type=jax.ShapeDtypeStruct((NUM_WORKERS, L, D), jnp.float32),
      mesh=mesh,
      scratch_types=scratch,
      compiler_params=pltpu.CompilerParams(use_tc_tiling_on_sc=False),
  )
  def body(ids_hbm, table_hbm, out_hbm, *refs):
    idx_v = refs[0]
    rows = refs[1:1 + NBUF]
    acc_v = refs[1 + NBUF]
    sems = refs[2 + NBUF:2 + 2 * NBUF]

    wid = lax.axis_index("s") * NUM_CORES + lax.axis_index("c")
    start = wid * ids_per_worker

    pltpu.sync_copy(ids_hbm.at[pl.ds(start, ids_per_worker)], idx_v)

    zero = jnp.zeros((LANES,), jnp.float32)
    for r in range(L):
      for h in range(vecs_per_row):
        acc_v[r, pl.ds(h * LANES, LANES)] = zero

    def chunk_idx(c):
      return idx_v.at[pl.ds(c * CHUNK, CHUNK)]

    for b in range(NBUF):
      pltpu.async_copy(table_hbm.at[chunk_idx(b)], rows[b], sems[b])

    def loop_body(it, carry):
      j = it * NBUF
      for b in range(NBUF):
        cur = j + b
        pltpu.make_async_copy(
            table_hbm.at[chunk_idx(cur)], rows[b], sems[b]).wait()
        l_dyn = (start + cur * CHUNK) >> log2_b
        accs = [zero] * (2 * vecs_per_row)
        for r in range(CHUNK):
          for h in range(vecs_per_row):
            a = (r % 2) * vecs_per_row + h
            accs[a] = accs[a] + rows[b][r, pl.ds(h * LANES, LANES)]
        for h in range(vecs_per_row):
          plsc.addupdate(acc_v.at[l_dyn, pl.ds(h * LANES, LANES)],
                         accs[h] + accs[vecs_per_row + h])
        nxt = cur + NBUF

        @pl.when(nxt < num_chunks)
        def _():
          pltpu.async_copy(table_hbm.at[chunk_idx(nxt)], rows[b], sems[b])
      return carry

    lax.fori_loop(0, num_chunks // NBUF, loop_body, 0, unroll=False)

    pltpu.sync_copy(acc_v, out_hbm.at[wid])

  return body(ids_flat, table)


def _tc_depad(table, V, D):
  R = 2000
  ng = V // (4 * R)

  def body(x0, x1, x2, x3, o_ref):
    o_ref[...] = jnp.concatenate(
        [x0[...], x1[...], x2[...], x3[...]], axis=1)

  specs = [
      pl.BlockSpec((R, D), (lambda g, u=u: (u * ng + g, 0)))
      for u in range(4)
  ]
  return pl.pallas_call(
      body,
      grid=(ng,),
      in_specs=specs,
      out_specs=pl.BlockSpec((R, 4 * D), lambda g: (g, 0)),
      out_shape=jax.ShapeDtypeStruct((V // 4, 4 * D), jnp.float32),
  )(table, table, table, table)


def _tc_combine(partials, L, D):
  def body(x_ref, o_ref):
    o_ref[...] = jnp.sum(x_ref[...], axis=0)

  return pl.pallas_call(
      body,
      out_shape=jax.ShapeDtypeStruct((L, D), jnp.float32),
  )(partials)


def kernel(ingredient_ids, table):
  B, L = ingredient_ids.shape
  V, D = table.shape
  ids = ingredient_ids.astype(jnp.int32)

  ids_per_worker = (B * L) // NUM_WORKERS
  assert (B * L) % (NUM_WORKERS * CHUNK) == 0
  assert B % CHUNK == 0 and D % LANES == 0

  ids_flat = ids.T.reshape(-1)
  tbl_lin = _tc_depad(table, V, D).reshape(V, D)
  quarter = ids_flat // (V // 4)
  ids_m = ids_flat * 4 - quarter * (V - 1)
  partials = _sc_partial_sums(ids_m, tbl_lin, B, L, D)
  return _tc_combine(partials, L, D)


# speedup vs baseline: 1.0198x; 1.0198x over previous
import functools

import jax
import jax.numpy as jnp
from jax import lax
from jax.experimental import pallas as pl
from jax.experimental.pallas import tpu as pltpu
from jax.experimental.pallas import tpu_sc as plsc

NUM_CORES = 2
NUM_SUBCORES = 16
NUM_WORKERS = NUM_CORES * NUM_SUBCORES
LANES = 16

CHUNK = 128
NBUF = 4


def _sc_partial_sums(ids_flat, table, B, L, D):
  ids_per_worker = (B * L) // NUM_WORKERS
  num_chunks = ids_per_worker // CHUNK
  vecs_per_row = D // LANES
  log2_b = B.bit_length() - 1
  assert (1 << log2_b) == B

  mesh = plsc.VectorSubcoreMesh(
      core_axis_name="c", subcore_axis_name="s",
      num_cores=NUM_CORES, num_subcores=NUM_SUBCORES)

  scratch = (
      [pltpu.VMEM((ids_per_worker,), jnp.int32)]
      + [pltpu.VMEM((CHUNK, D), jnp.float32) for _ in range(NBUF)]
      + [pltpu.VMEM((L, D), jnp.float32)]
      + [pltpu.SemaphoreType.DMA for _ in range(NBUF)]
  )

  @functools.partial(
      pl.kernel,
      out_type=jax.ShapeDtypeStruct((NUM_WORKERS, L, D), jnp.float32),
      mesh=mesh,
      scratch_types=scratch,
      compiler_params=pltpu.CompilerParams(use_tc_tiling_on_sc=False),
  )
  def body(ids_hbm, table_hbm, out_hbm, *refs):
    idx_v = refs[0]
    rows = refs[1:1 + NBUF]
    acc_v = refs[1 + NBUF]
    sems = refs[2 + NBUF:2 + 2 * NBUF]

    wid = lax.axis_index("s") * NUM_CORES + lax.axis_index("c")
    start = wid * ids_per_worker

    pltpu.sync_copy(ids_hbm.at[pl.ds(start, ids_per_worker)], idx_v)

    zero = jnp.zeros((LANES,), jnp.float32)
    for r in range(L):
      for h in range(vecs_per_row):
        acc_v[r, pl.ds(h * LANES, LANES)] = zero

    def chunk_idx(c):
      return idx_v.at[pl.ds(c * CHUNK, CHUNK)]

    for b in range(NBUF):
      pltpu.async_copy(table_hbm.at[chunk_idx(b)], rows[b], sems[b])

    def loop_body(it, carry):
      j = it * NBUF
      for b in range(NBUF):
        cur = j + b
        pltpu.make_async_copy(
            table_hbm.at[chunk_idx(cur)], rows[b], sems[b]).wait()
        l_dyn = (start + cur * CHUNK) >> log2_b
        accs = [zero] * (2 * vecs_per_row)
        for r in range(CHUNK):
          for h in range(vecs_per_row):
            a = (r % 2) * vecs_per_row + h
            accs[a] = accs[a] + rows[b][r, pl.ds(h * LANES, LANES)]
        for h in range(vecs_per_row):
          plsc.addupdate(acc_v.at[l_dyn, pl.ds(h * LANES, LANES)],
                         accs[h] + accs[vecs_per_row + h])
        nxt = cur + NBUF

        @pl.when(nxt < num_chunks)
        def _():
          pltpu.async_copy(table_hbm.at[chunk_idx(nxt)], rows[b], sems[b])
      return carry

    lax.fori_loop(0, num_chunks // NBUF, loop_body, 0, unroll=False)

    pltpu.sync_copy(acc_v, out_hbm.at[wid])

  return body(ids_flat, table)


def _tc_depad(table, V, D):
  R = 2000
  ng = V // (4 * R)

  def body(x_ref, o_ref):
    x = x_ref[...]
    o_ref[...] = jnp.concatenate(
        [x[u * R:(u + 1) * R, :] for u in range(4)], axis=1)

  return pl.pallas_call(
      body,
      grid=(ng,),
      in_specs=[pl.BlockSpec((4 * R, D), lambda g: (g, 0))],
      out_specs=pl.BlockSpec((R, 4 * D), lambda g: (g, 0)),
      out_shape=jax.ShapeDtypeStruct((V // 4, 4 * D), jnp.float32),
  )(table)


def _tc_combine(partials, L, D):
  def body(x_ref, o_ref):
    o_ref[...] = jnp.sum(x_ref[...], axis=0)

  return pl.pallas_call(
      body,
      out_shape=jax.ShapeDtypeStruct((L, D), jnp.float32),
  )(partials)


def kernel(ingredient_ids, table):
  B, L = ingredient_ids.shape
  V, D = table.shape
  ids = ingredient_ids.astype(jnp.int32)

  ids_per_worker = (B * L) // NUM_WORKERS
  assert (B * L) % (NUM_WORKERS * CHUNK) == 0
  assert B % CHUNK == 0 and D % LANES == 0

  ids_flat = ids.T.reshape(-1)
  tbl_lin = _tc_depad(table, V, D).reshape(V, D)
  R = 2000
  blk = ids_flat // (4 * R)
  rem = ids_flat - blk * (4 * R)
  u = rem // R
  s = rem - u * R
  ids_m = (blk * R + s) * 4 + u
  partials = _sc_partial_sums(ids_m, tbl_lin, B, L, D)
  return _tc_combine(partials, L, D)


# speedup vs baseline: 2.6821x; 2.6299x over previous
import functools

import jax
import jax.numpy as jnp
from jax import lax
from jax.experimental import pallas as pl
from jax.experimental.pallas import tpu as pltpu
from jax.experimental.pallas import tpu_sc as plsc

NUM_CORES = 2
NUM_SUBCORES = 16
NUM_WORKERS = NUM_CORES * NUM_SUBCORES
LANES = 16

CHUNK = 128
NBUF = 4


def _sc_partial_sums(ids_flat, table, B, L, D):
  ids_per_worker = (B * L) // NUM_WORKERS
  num_chunks = ids_per_worker // CHUNK
  vecs_per_row = D // LANES
  log2_b = B.bit_length() - 1
  assert (1 << log2_b) == B

  mesh = plsc.VectorSubcoreMesh(
      core_axis_name="c", subcore_axis_name="s",
      num_cores=NUM_CORES, num_subcores=NUM_SUBCORES)

  scratch = (
      [pltpu.VMEM((ids_per_worker,), jnp.int32)]
      + [pltpu.VMEM((CHUNK, D), jnp.float32) for _ in range(NBUF)]
      + [pltpu.VMEM((L, D), jnp.float32)]
      + [pltpu.SemaphoreType.DMA for _ in range(NBUF)]
  )

  @functools.partial(
      pl.kernel,
      out_type=jax.ShapeDtypeStruct((NUM_WORKERS, L, D), jnp.float32),
      mesh=mesh,
      scratch_types=scratch,
      compiler_params=pltpu.CompilerParams(use_tc_tiling_on_sc=False),
  )
  def body(ids_hbm, table_hbm, out_hbm, *refs):
    idx_v = refs[0]
    rows = refs[1:1 + NBUF]
    acc_v = refs[1 + NBUF]
    sems = refs[2 + NBUF:2 + 2 * NBUF]

    wid = lax.axis_index("s") * NUM_CORES + lax.axis_index("c")
    start = wid * ids_per_worker

    pltpu.sync_copy(ids_hbm.at[pl.ds(start, ids_per_worker)], idx_v)

    zero = jnp.zeros((LANES,), jnp.float32)
    for r in range(L):
      for h in range(vecs_per_row):
        acc_v[r, pl.ds(h * LANES, LANES)] = zero

    def chunk_idx(c):
      return idx_v.at[pl.ds(c * CHUNK, CHUNK)]

    for b in range(NBUF):
      pltpu.async_copy(table_hbm.at[chunk_idx(b)], rows[b], sems[b])

    def loop_body(it, carry):
      j = it * NBUF
      for b in range(NBUF):
        cur = j + b
        pltpu.make_async_copy(
            table_hbm.at[chunk_idx(cur)], rows[b], sems[b]).wait()
        l_dyn = (start + cur * CHUNK) >> log2_b
        accs = [zero] * (2 * vecs_per_row)
        for r in range(CHUNK):
          for h in range(vecs_per_row):
            a = (r % 2) * vecs_per_row + h
            accs[a] = accs[a] + rows[b][r, pl.ds(h * LANES, LANES)]
        for h in range(vecs_per_row):
          plsc.addupdate(acc_v.at[l_dyn, pl.ds(h * LANES, LANES)],
                         accs[h] + accs[vecs_per_row + h])
        nxt = cur + NBUF

        @pl.when(nxt < num_chunks)
        def _():
          pltpu.async_copy(table_hbm.at[chunk_idx(nxt)], rows[b], sems[b])
      return carry

    lax.fori_loop(0, num_chunks // NBUF, loop_body, 0, unroll=False)

    pltpu.sync_copy(acc_v, out_hbm.at[wid])

  return body(ids_flat, table)


def _tc_depad(table, V, D):
  R = 2048
  ng = -(-V // (4 * R))
  tableT = table.T

  def body(x_ref, o_ref):
    x = x_ref[...]
    y = jnp.concatenate([x[:, u * R:(u + 1) * R] for u in range(4)], axis=0)
    o_ref[...] = y.T

  return pl.pallas_call(
      body,
      grid=(ng,),
      in_specs=[pl.BlockSpec((D, 4 * R), lambda g: (0, g))],
      out_specs=pl.BlockSpec((R, 4 * D), lambda g: (g, 0)),
      out_shape=jax.ShapeDtypeStruct((ng * R, 4 * D), jnp.float32),
  )(tableT)


def _tc_combine(partials, L, D):
  def body(x_ref, o_ref):
    o_ref[...] = jnp.sum(x_ref[...], axis=0)

  return pl.pallas_call(
      body,
      out_shape=jax.ShapeDtypeStruct((L, D), jnp.float32),
  )(partials)


def kernel(ingredient_ids, table):
  B, L = ingredient_ids.shape
  V, D = table.shape
  ids = ingredient_ids.astype(jnp.int32)

  ids_per_worker = (B * L) // NUM_WORKERS
  assert (B * L) % (NUM_WORKERS * CHUNK) == 0
  assert B % CHUNK == 0 and D % LANES == 0

  ids_flat = ids.T.reshape(-1)
  tbl4 = _tc_depad(table, V, D)
  vp = tbl4.shape[0] * 4
  tbl_lin = tbl4.reshape(vp, D)
  R = 2048
  blk = ids_flat >> 13
  rem = ids_flat & (4 * R - 1)
  u = rem >> 11
  s = rem & (R - 1)
  ids_m = (blk * R + s) * 4 + u
  partials = _sc_partial_sums(ids_m, tbl_lin, B, L, D)
  return _tc_combine(partials, L, D)


# speedup vs baseline: 3.4588x; 1.2896x over previous
import functools

import jax
import jax.numpy as jnp
from jax import lax
from jax.experimental import pallas as pl
from jax.experimental.pallas import tpu as pltpu
from jax.experimental.pallas import tpu_sc as plsc

NUM_CORES = 2
NUM_SUBCORES = 16
NUM_WORKERS = NUM_CORES * NUM_SUBCORES
LANES = 16

CHUNK = 128
NBUF = 4
REPACK_R = 16384


def _sc_partial_sums(ids_flat, table, B, L, D):
  ids_per_worker = (B * L) // NUM_WORKERS
  num_chunks = ids_per_worker // CHUNK
  vecs_per_row = D // LANES
  log2_b = B.bit_length() - 1
  assert (1 << log2_b) == B

  mesh = plsc.VectorSubcoreMesh(
      core_axis_name="c", subcore_axis_name="s",
      num_cores=NUM_CORES, num_subcores=NUM_SUBCORES)

  scratch = (
      [pltpu.VMEM((ids_per_worker,), jnp.int32)]
      + [pltpu.VMEM((CHUNK, D), jnp.float32) for _ in range(NBUF)]
      + [pltpu.VMEM((L, D), jnp.float32)]
      + [pltpu.SemaphoreType.DMA for _ in range(NBUF)]
  )

  @functools.partial(
      pl.kernel,
      out_type=jax.ShapeDtypeStruct((NUM_WORKERS, L, D), jnp.float32),
      mesh=mesh,
      scratch_types=scratch,
      compiler_params=pltpu.CompilerParams(use_tc_tiling_on_sc=False),
  )
  def body(ids_hbm, table_hbm, out_hbm, *refs):
    idx_v = refs[0]
    rows = refs[1:1 + NBUF]
    acc_v = refs[1 + NBUF]
    sems = refs[2 + NBUF:2 + 2 * NBUF]

    wid = lax.axis_index("s") * NUM_CORES + lax.axis_index("c")
    start = wid * ids_per_worker

    pltpu.sync_copy(ids_hbm.at[pl.ds(start, ids_per_worker)], idx_v)

    zero = jnp.zeros((LANES,), jnp.float32)
    for r in range(L):
      for h in range(vecs_per_row):
        acc_v[r, pl.ds(h * LANES, LANES)] = zero

    def chunk_idx(c):
      return idx_v.at[pl.ds(c * CHUNK, CHUNK)]

    for b in range(NBUF):
      pltpu.async_copy(table_hbm.at[chunk_idx(b)], rows[b], sems[b])

    def loop_body(it, carry):
      j = it * NBUF
      for b in range(NBUF):
        cur = j + b
        pltpu.make_async_copy(
            table_hbm.at[chunk_idx(cur)], rows[b], sems[b]).wait()
        l_dyn = (start + cur * CHUNK) >> log2_b
        accs = [zero] * (2 * vecs_per_row)
        for r in range(CHUNK):
          for h in range(vecs_per_row):
            a = (r % 2) * vecs_per_row + h
            accs[a] = accs[a] + rows[b][r, pl.ds(h * LANES, LANES)]
        for h in range(vecs_per_row):
          plsc.addupdate(acc_v.at[l_dyn, pl.ds(h * LANES, LANES)],
                         accs[h] + accs[vecs_per_row + h])
        nxt = cur + NBUF

        @pl.when(nxt < num_chunks)
        def _():
          pltpu.async_copy(table_hbm.at[chunk_idx(nxt)], rows[b], sems[b])
      return carry

    lax.fori_loop(0, num_chunks // NBUF, loop_body, 0, unroll=False)

    pltpu.sync_copy(acc_v, out_hbm.at[wid])

  return body(ids_flat, table)


def _tc_depad(table, V, D):
  R = REPACK_R
  ng = -(-V // (4 * R))
  tableT = table.T

  def body(x_ref, o_ref):
    x = x_ref[...]
    y = jnp.concatenate([x[:, u * R:(u + 1) * R] for u in range(4)], axis=0)
    o_ref[...] = y.T

  return pl.pallas_call(
      body,
      grid=(ng,),
      in_specs=[pl.BlockSpec((D, 4 * R), lambda g: (0, g))],
      out_specs=pl.BlockSpec((R, 4 * D), lambda g: (g, 0)),
      out_shape=jax.ShapeDtypeStruct((ng * R, 4 * D), jnp.float32),
  )(tableT)


def _tc_combine(partials, L, D):
  def body(x_ref, o_ref):
    o_ref[...] = jnp.sum(x_ref[...], axis=0)

  return pl.pallas_call(
      body,
      out_shape=jax.ShapeDtypeStruct((L, D), jnp.float32),
  )(partials)


def kernel(ingredient_ids, table):
  B, L = ingredient_ids.shape
  V, D = table.shape
  ids = ingredient_ids.astype(jnp.int32)

  ids_per_worker = (B * L) // NUM_WORKERS
  assert (B * L) % (NUM_WORKERS * CHUNK) == 0
  assert B % CHUNK == 0 and D % LANES == 0

  ids_flat = ids.T.reshape(-1)
  tbl4 = _tc_depad(table, V, D)
  vp = tbl4.shape[0] * 4
  tbl_lin = tbl4.reshape(vp, D)
  R = REPACK_R
  log2_r = R.bit_length() - 1
  blk = ids_flat >> (log2_r + 2)
  rem = ids_flat & (4 * R - 1)
  u = rem >> log2_r
  s = rem & (R - 1)
  ids_m = (blk * R + s) * 4 + u
  partials = _sc_partial_sums(ids_m, tbl_lin, B, L, D)
  return _tc_combine(partials, L, D)
